# Initial kernel scaffold; baseline (speedup 1.0000x reference)
import jax
import jax.numpy as jnp
from jax.experimental import pallas as pl


def kernel(x):
    raise NotImplementedError("write your pallas kernel here")


# speedup vs baseline: 18.1809x; 18.1809x over previous
import functools

import jax
import jax.numpy as jnp
from jax import lax
from jax.experimental import pallas as pl
from jax.experimental.pallas import tpu as pltpu

_SR = 0.5


def _sparsify_block(x_ref, o_ref, *, k):
    x = x_ref[...]
    y = lax.bitcast_convert_type(x, jnp.uint32)
    sign = jnp.uint32(0x80000000)
    neg = y >= sign
    ukey = jnp.where(neg, ~y, y ^ sign)

    rows = x.shape[0]
    u = jnp.zeros((rows, 1), dtype=jnp.uint32)
    for b in range(31, -1, -1):
        cand = u | jnp.uint32(1 << b)
        cnt = jnp.sum((ukey >= cand).astype(jnp.int32), axis=1, keepdims=True)
        u = jnp.where(cnt >= k, cand, u)

    tbits = jnp.where(u >= sign, u ^ sign, ~u)
    t = lax.bitcast_convert_type(tbits, jnp.float32)
    o_ref[...] = jnp.where(x >= t, x, jnp.float32(0.0))


@jax.jit
def kernel(x):
    rows, cols = x.shape
    k = int(_SR * cols)
    blk = 16
    grid = (rows // blk,)
    return pl.pallas_call(
        functools.partial(_sparsify_block, k=k),
        grid=grid,
        in_specs=[pl.BlockSpec((blk, cols), lambda i: (i, 0))],
        out_specs=pl.BlockSpec((blk, cols), lambda i: (i, 0)),
        out_shape=jax.ShapeDtypeStruct((rows, cols), x.dtype),
    )(x)
